# Initial kernel scaffold; baseline (speedup 1.0000x reference)
import jax
import jax.numpy as jnp
from jax.experimental import pallas as pl


def kernel(x, edge_index, W1z, b1z, W1r, b1r, W1h, b1h, W2z, b2z, W2r, b2r, W2h, b2h):
    raise NotImplementedError("write your pallas kernel here")


# speedup vs baseline: 4.1844x; 4.1844x over previous
import jax
import jax.numpy as jnp
from jax.experimental import pallas as pl

_N = 10000
_IN = 256
_OUT = 128
_H1 = 2 * _OUT
_TILE = 2000


def _fused_encoder(x_ref, w1z_ref, b1z_ref, w1h_ref, b1h_ref,
                   w2z_ref, b2z_ref, w2h_ref, b2h_ref, o_ref):
    x = x_ref[...]
    z1 = jax.nn.sigmoid(
        jnp.dot(x, w1z_ref[...], preferred_element_type=jnp.float32) + b1z_ref[...])
    h1 = jnp.tanh(
        jnp.dot(x, w1h_ref[...], preferred_element_type=jnp.float32) + b1h_ref[...])
    h = jnp.maximum((1.0 - z1) * h1, 0.0)
    z2 = jax.nn.sigmoid(
        jnp.dot(h, w2z_ref[...], preferred_element_type=jnp.float32) + b2z_ref[...])
    h2 = jnp.tanh(
        jnp.dot(h, w2h_ref[...], preferred_element_type=jnp.float32) + b2h_ref[...])
    o_ref[...] = (1.0 - z2) * h2


def kernel(x, edge_index, W1z, b1z, W1r, b1r, W1h, b1h,
           W2z, b2z, W2r, b2r, W2h, b2h):
    del edge_index, W1r, b1r, W2r, b2r
    w1z = W1z[0, 0, :_IN] + W1z[1, 0, :_IN]
    w1h = W1h[0, 0, :_IN] + W1h[1, 0, :_IN]
    w2z = W2z[0, 0, :_H1] + W2z[1, 0, :_H1]
    w2h = W2h[0, 0, :_H1] + W2h[1, 0, :_H1]

    grid = _N // _TILE
    row_spec = pl.BlockSpec((_TILE, _IN), lambda i: (i, 0))
    full = lambda shape: pl.BlockSpec(shape, lambda i: (0,) * len(shape))

    return pl.pallas_call(
        _fused_encoder,
        grid=(grid,),
        in_specs=[
            row_spec,
            full((_IN, _H1)), full((1, _H1)),
            full((_IN, _H1)), full((1, _H1)),
            full((_H1, _OUT)), full((1, _OUT)),
            full((_H1, _OUT)), full((1, _OUT)),
        ],
        out_specs=pl.BlockSpec((_TILE, _OUT), lambda i: (i, 0)),
        out_shape=jax.ShapeDtypeStruct((_N, _OUT), jnp.float32),
    )(x, w1z, b1z.reshape(1, _H1), w1h, b1h.reshape(1, _H1),
      w2z, b2z.reshape(1, _OUT), w2h, b2h.reshape(1, _OUT))


# speedup vs baseline: 4.2044x; 1.0048x over previous
import jax
import jax.numpy as jnp
from jax.experimental import pallas as pl
from jax.experimental.pallas import tpu as pltpu

_N = 10000
_IN = 256
_OUT = 128
_H1 = 2 * _OUT
_TILE = 2000


def _fused_encoder(x_ref, w1z_ref, b1z_ref, w1h_ref, b1h_ref,
                   w2z_ref, b2z_ref, w2h_ref, b2h_ref, o_ref):
    x = x_ref[...]
    z1 = jax.nn.sigmoid(
        jnp.dot(x, w1z_ref[...], preferred_element_type=jnp.float32) + b1z_ref[...])
    h1 = jnp.tanh(
        jnp.dot(x, w1h_ref[...], preferred_element_type=jnp.float32) + b1h_ref[...])
    h = jnp.maximum((1.0 - z1) * h1, 0.0)
    z2 = jax.nn.sigmoid(
        jnp.dot(h, w2z_ref[...], preferred_element_type=jnp.float32) + b2z_ref[...])
    h2 = jnp.tanh(
        jnp.dot(h, w2h_ref[...], preferred_element_type=jnp.float32) + b2h_ref[...])
    o_ref[...] = (1.0 - z2) * h2


def kernel(x, edge_index, W1z, b1z, W1r, b1r, W1h, b1h,
           W2z, b2z, W2r, b2r, W2h, b2h):
    del edge_index, W1r, b1r, W2r, b2r
    w1z = W1z[0, 0, :_IN] + W1z[1, 0, :_IN]
    w1h = W1h[0, 0, :_IN] + W1h[1, 0, :_IN]
    w2z = W2z[0, 0, :_H1] + W2z[1, 0, :_H1]
    w2h = W2h[0, 0, :_H1] + W2h[1, 0, :_H1]

    grid = _N // _TILE
    row_spec = pl.BlockSpec((_TILE, _IN), lambda i: (i, 0))
    full = lambda shape: pl.BlockSpec(shape, lambda i: (0,) * len(shape))

    return pl.pallas_call(
        _fused_encoder,
        grid=(grid,),
        in_specs=[
            row_spec,
            full((_IN, _H1)), full((1, _H1)),
            full((_IN, _H1)), full((1, _H1)),
            full((_H1, _OUT)), full((1, _OUT)),
            full((_H1, _OUT)), full((1, _OUT)),
        ],
        out_specs=pl.BlockSpec((_TILE, _OUT), lambda i: (i, 0)),
        out_shape=jax.ShapeDtypeStruct((_N, _OUT), jnp.float32),
        compiler_params=pltpu.CompilerParams(
            dimension_semantics=("parallel",)),
    )(x, w1z, b1z.reshape(1, _H1), w1h, b1h.reshape(1, _H1),
      w2z, b2z.reshape(1, _OUT), w2h, b2h.reshape(1, _OUT))
